# Initial kernel scaffold; baseline (speedup 1.0000x reference)
import jax
import jax.numpy as jnp
from jax.experimental import pallas as pl


def kernel(x, adjx, W1, b1, W2, b2, gn_gamma, gn_beta, fc3_W, fc3_b):
    raise NotImplementedError("write your pallas kernel here")


# speedup vs baseline: 1.0677x; 1.0677x over previous
import functools

import jax
import jax.numpy as jnp
from jax.experimental import pallas as pl
from jax.experimental.pallas import tpu as pltpu


def _linear_kernel(x_ref, w_ref, o_ref):
    o_ref[:] = jnp.dot(x_ref[:], w_ref[:], preferred_element_type=jnp.float32)


def _layer_kernel(adj_ref, s_ref, b_ref, g_ref, bt_ref, w_ref, wb_ref, o_ref):
    y = jnp.dot(adj_ref[:], s_ref[:], preferred_element_type=jnp.float32)
    h = y + b_ref[:]
    mean = jnp.mean(h, axis=1, keepdims=True)
    var = jnp.mean((h - mean) ** 2, axis=1, keepdims=True)
    h = (h - mean) * jax.lax.rsqrt(var + 1e-5)
    h = h * g_ref[:] + bt_ref[:]
    h = jnp.where(h >= 0, h, 0.01 * h)
    o_ref[:] = jnp.dot(h, w_ref[:], preferred_element_type=jnp.float32) + wb_ref[:]


def _layer(adj, s, b, gamma, beta, w, wb, block_m):
    n, _ = adj.shape
    d = s.shape[1]
    do = w.shape[1]
    grid = (n // block_m,)
    return pl.pallas_call(
        _layer_kernel,
        grid=grid,
        in_specs=[
            pl.BlockSpec((block_m, n), lambda i: (i, 0)),
            pl.BlockSpec((n, d), lambda i: (0, 0)),
            pl.BlockSpec((1, d), lambda i: (0, 0)),
            pl.BlockSpec((1, d), lambda i: (0, 0)),
            pl.BlockSpec((1, d), lambda i: (0, 0)),
            pl.BlockSpec((d, do), lambda i: (0, 0)),
            pl.BlockSpec((1, do), lambda i: (0, 0)),
        ],
        out_specs=pl.BlockSpec((block_m, do), lambda i: (i, 0)),
        out_shape=jax.ShapeDtypeStruct((n, do), jnp.float32),
        compiler_params=pltpu.CompilerParams(
            dimension_semantics=("arbitrary",),
        ),
    )(adj, s, b, gamma, beta, w, wb)


def kernel(x, adjx, W1, b1, W2, b2, gn_gamma, gn_beta, fc3_W, fc3_b):
    n, d_in = x.shape
    d_h = W1.shape[1]
    d_out = fc3_W.shape[0]
    block_m = 200

    row = lambda v: v.reshape(1, -1)
    zeros_h = jnp.zeros((1, d_h), dtype=jnp.float32)

    s1 = pl.pallas_call(
        _linear_kernel,
        out_shape=jax.ShapeDtypeStruct((n, d_h), jnp.float32),
    )(x, W1)

    s2 = _layer(adjx, s1, row(b1), row(gn_gamma), row(gn_beta), W2, zeros_h,
                block_m)
    out = _layer(adjx, s2, row(b2), row(gn_gamma), row(gn_beta), fc3_W.T,
                 row(fc3_b), block_m)
    return out


# speedup vs baseline: 1.1201x; 1.0490x over previous
import functools

import jax
import jax.numpy as jnp
from jax.experimental import pallas as pl
from jax.experimental.pallas import tpu as pltpu


def _linear_kernel(x_ref, w_ref, o_ref):
    o_ref[:] = jnp.dot(x_ref[:], w_ref[:], preferred_element_type=jnp.float32)


def _layer_kernel(adj_ref, s_ref, b_ref, g_ref, bt_ref, w_ref, wb_ref, o_ref):
    y = jnp.dot(adj_ref[:], s_ref[:], preferred_element_type=jnp.float32)
    h = y + b_ref[:]
    mean = jnp.mean(h, axis=1, keepdims=True)
    var = jnp.mean((h - mean) ** 2, axis=1, keepdims=True)
    h = (h - mean) * jax.lax.rsqrt(var + 1e-5)
    h = h * g_ref[:] + bt_ref[:]
    h = jnp.where(h >= 0, h, 0.01 * h)
    o_ref[:] = jnp.dot(h, w_ref[:], preferred_element_type=jnp.float32) + wb_ref[:]


def _layer(adj, s, b, gamma, beta, w, wb, block_m):
    n, _ = adj.shape
    d = s.shape[1]
    do = w.shape[1]
    grid = (n // block_m,)
    return pl.pallas_call(
        _layer_kernel,
        grid=grid,
        in_specs=[
            pl.BlockSpec((block_m, n), lambda i: (i, 0)),
            pl.BlockSpec((n, d), lambda i: (0, 0)),
            pl.BlockSpec((1, d), lambda i: (0, 0)),
            pl.BlockSpec((1, d), lambda i: (0, 0)),
            pl.BlockSpec((1, d), lambda i: (0, 0)),
            pl.BlockSpec((d, do), lambda i: (0, 0)),
            pl.BlockSpec((1, do), lambda i: (0, 0)),
        ],
        out_specs=pl.BlockSpec((block_m, do), lambda i: (i, 0)),
        out_shape=jax.ShapeDtypeStruct((n, do), jnp.float32),
        compiler_params=pltpu.CompilerParams(
            dimension_semantics=("arbitrary",),
        ),
    )(adj, s, b, gamma, beta, w, wb)


def kernel(x, adjx, W1, b1, W2, b2, gn_gamma, gn_beta, fc3_W, fc3_b):
    n, d_in = x.shape
    d_h = W1.shape[1]
    d_out = fc3_W.shape[0]
    block_m = 400

    row = lambda v: v.reshape(1, -1)
    zeros_h = jnp.zeros((1, d_h), dtype=jnp.float32)

    s1 = pl.pallas_call(
        _linear_kernel,
        out_shape=jax.ShapeDtypeStruct((n, d_h), jnp.float32),
    )(x, W1)

    s2 = _layer(adjx, s1, row(b1), row(gn_gamma), row(gn_beta), W2, zeros_h,
                block_m)
    out = _layer(adjx, s2, row(b2), row(gn_gamma), row(gn_beta), fc3_W.T,
                 row(fc3_b), block_m)
    return out


# speedup vs baseline: 1.1216x; 1.0014x over previous
import functools

import jax
import jax.numpy as jnp
from jax.experimental import pallas as pl
from jax.experimental.pallas import tpu as pltpu


def _linear_kernel(x_ref, w_ref, o_ref):
    o_ref[:] = jnp.dot(x_ref[:], w_ref[:], preferred_element_type=jnp.float32)


def _layer_kernel(adj_ref, s_ref, b_ref, g_ref, bt_ref, w_ref, wb_ref, o_ref):
    y = jnp.dot(adj_ref[:].astype(jnp.bfloat16), s_ref[:].astype(jnp.bfloat16),
                preferred_element_type=jnp.float32)
    h = y + b_ref[:]
    mean = jnp.mean(h, axis=1, keepdims=True)
    var = jnp.mean((h - mean) ** 2, axis=1, keepdims=True)
    h = (h - mean) * jax.lax.rsqrt(var + 1e-5)
    h = h * g_ref[:] + bt_ref[:]
    h = jnp.where(h >= 0, h, 0.01 * h)
    o_ref[:] = jnp.dot(h, w_ref[:], preferred_element_type=jnp.float32) + wb_ref[:]


def _layer(adj, s, b, gamma, beta, w, wb, block_m):
    n, _ = adj.shape
    d = s.shape[1]
    do = w.shape[1]
    grid = (n // block_m,)
    return pl.pallas_call(
        _layer_kernel,
        grid=grid,
        in_specs=[
            pl.BlockSpec((block_m, n), lambda i: (i, 0)),
            pl.BlockSpec((n, d), lambda i: (0, 0)),
            pl.BlockSpec((1, d), lambda i: (0, 0)),
            pl.BlockSpec((1, d), lambda i: (0, 0)),
            pl.BlockSpec((1, d), lambda i: (0, 0)),
            pl.BlockSpec((d, do), lambda i: (0, 0)),
            pl.BlockSpec((1, do), lambda i: (0, 0)),
        ],
        out_specs=pl.BlockSpec((block_m, do), lambda i: (i, 0)),
        out_shape=jax.ShapeDtypeStruct((n, do), jnp.float32),
        compiler_params=pltpu.CompilerParams(
            dimension_semantics=("arbitrary",),
        ),
    )(adj, s, b, gamma, beta, w, wb)


def kernel(x, adjx, W1, b1, W2, b2, gn_gamma, gn_beta, fc3_W, fc3_b):
    n, d_in = x.shape
    d_h = W1.shape[1]
    d_out = fc3_W.shape[0]
    block_m = 400

    row = lambda v: v.reshape(1, -1)
    zeros_h = jnp.zeros((1, d_h), dtype=jnp.float32)

    s1 = pl.pallas_call(
        _linear_kernel,
        out_shape=jax.ShapeDtypeStruct((n, d_h), jnp.float32),
    )(x, W1)

    s2 = _layer(adjx, s1, row(b1), row(gn_gamma), row(gn_beta), W2, zeros_h,
                block_m)
    out = _layer(adjx, s2, row(b2), row(gn_gamma), row(gn_beta), fc3_W.T,
                 row(fc3_b), block_m)
    return out
